# baseline (speedup 1.0000x reference)
import functools

import jax
import jax.numpy as jnp
from jax import lax
from jax.experimental import pallas as pl
from jax.experimental.pallas import tpu as pltpu
from jax.experimental.pallas import tpu_sc as plsc

_B, _T, _D, _E, _K = 1, 2048, 768, 8, 2
_H = 4 * _D
_NP = _T * _K
_TR = 128
_NR = _NP + _E * _TR
_G = _NR // _TR
_NW = 32
_CW = _T // _NW
_SUB = 32


def _erf(z):
    s = jnp.sign(z)
    a = jnp.abs(z)
    t = 1.0 / (1.0 + 0.3275911 * a)
    p = t * (0.254829592 + t * (-0.284496736 + t * (1.421413741
        + t * (-1.453152027 + t * 1.061405429))))
    return s * (1.0 - p * jnp.exp(-a * a))


def _gelu(x):
    return 0.5 * x * (1.0 + _erf(x * 0.7071067811865476))


def _cumsum_rows(a):
    n = a.shape[1]
    sh = 1
    while sh < n:
        z = jnp.zeros(a.shape[:1] + (sh,), a.dtype)
        a = a + jnp.concatenate([z, a[:, :-sh]], axis=1)
        sh *= 2
    return a


def _route_body(idx_ref, d_ref, meta_ref):
    ee = lax.broadcasted_iota(jnp.int32, (_E, 1), 0)
    m0 = (idx_ref[0:1, :] == ee).astype(jnp.float32)
    m1 = (idx_ref[1:2, :] == ee).astype(jnp.float32)
    inc0 = _cumsum_rows(m0)
    inc1 = _cumsum_rows(m1) + inc0[:, _T - 1:_T]
    counts = inc1[:, _T - 1:_T]
    padded = jnp.ceil(counts * (1.0 / _TR)) * float(_TR)
    c = padded
    sh = 1
    while sh < _E:
        z = jnp.zeros((sh, 1), jnp.float32)
        c = c + jnp.concatenate([z, c[:-sh, :]], axis=0)
        sh *= 2
    starts = c - padded
    d0 = jnp.sum(m0 * (starts + inc0), axis=0, keepdims=True) - 1.0
    d1 = jnp.sum(m1 * (starts + inc1), axis=0, keepdims=True) - 1.0
    d_ref[...] = jnp.concatenate([d0, d1], axis=0).astype(jnp.int32)

    ends = starts + padded
    tpos = (lax.broadcasted_iota(jnp.int32, (1, _G), 1)
            .astype(jnp.float32) * float(_TR))
    neid = jnp.sum((tpos >= ends).astype(jnp.float32), axis=0, keepdims=True)
    eid = jnp.clip(neid, 0, _E - 1)
    prev = jnp.concatenate([jnp.full((1, 1), -1.0, jnp.float32),
                            eid[:, :-1]], axis=1)
    first = (eid != prev).astype(jnp.float32)
    runidx = _cumsum_rows(first) - 1.0
    parity = runidx - 2.0 * jnp.floor(runidx * 0.5)
    nruns = jnp.sum(first, axis=1, keepdims=True)
    rr = lax.broadcasted_iota(jnp.int32, (_G, 1), 0).astype(jnp.float32)
    is_start = first * (runidx + 1.0)
    reid = jnp.sum(jnp.where(rr + 1.0 == is_start, eid, 0.0), axis=1,
                   keepdims=True)
    nxt = jnp.sum(jnp.where(rr == runidx + 1.0, reid, 0.0), axis=0,
                  keepdims=True)
    nxt = jnp.where(runidx + 1.0 < nruns, nxt, -1.0)
    meta = jnp.concatenate([eid, first, parity, nxt], axis=0)
    meta_ref[...] = meta.astype(jnp.int32)


def _ffn_body(meta_sref, xs_ref, b1_ref, b2_ref, rw_ref, W1_any, W2_any,
              out_ref, w1a, w2a, w1b, w2b, sems):
    i = pl.program_id(0)
    eid = meta_sref[0, i]
    first = meta_sref[1, i] == 1
    par = meta_sref[2, i]
    nei = meta_sref[3, i]

    @pl.when(i == 0)
    def _():
        pltpu.make_async_copy(W1_any.at[eid], w1a, sems.at[0, 0]).start()
        pltpu.make_async_copy(W2_any.at[eid], w2a, sems.at[0, 1]).start()

    @pl.when(first)
    def _():
        @pl.when(par == 0)
        def _():
            pltpu.make_async_copy(W1_any.at[eid], w1a, sems.at[0, 0]).wait()
            pltpu.make_async_copy(W2_any.at[eid], w2a, sems.at[0, 1]).wait()

        @pl.when(par == 1)
        def _():
            pltpu.make_async_copy(W1_any.at[eid], w1b, sems.at[1, 0]).wait()
            pltpu.make_async_copy(W2_any.at[eid], w2b, sems.at[1, 1]).wait()

        @pl.when(nei >= 0)
        def _():
            @pl.when(par == 0)
            def _():
                pltpu.make_async_copy(W1_any.at[nei], w1b,
                                      sems.at[1, 0]).start()
                pltpu.make_async_copy(W2_any.at[nei], w2b,
                                      sems.at[1, 1]).start()

            @pl.when(par == 1)
            def _():
                pltpu.make_async_copy(W1_any.at[nei], w1a,
                                      sems.at[0, 0]).start()
                pltpu.make_async_copy(W2_any.at[nei], w2a,
                                      sems.at[0, 1]).start()

    xb = xs_ref[...]
    rw = rw_ref[:, 0:1]

    @pl.when(par == 0)
    def _():
        h = jnp.dot(xb, w1a[...], preferred_element_type=jnp.float32)
        h = _gelu(h + b1_ref[0])
        y = jnp.dot(h, w2a[...], preferred_element_type=jnp.float32)
        out_ref[...] = (y + b2_ref[0]) * rw

    @pl.when(par == 1)
    def _():
        h = jnp.dot(xb, w1b[...], preferred_element_type=jnp.float32)
        h = _gelu(h + b1_ref[0])
        y = jnp.dot(h, w2b[...], preferred_element_type=jnp.float32)
        out_ref[...] = (y + b2_ref[0]) * rw


def _make_dispatch():
    mesh = plsc.VectorSubcoreMesh(core_axis_name="c", subcore_axis_name="s")

    @functools.partial(
        pl.kernel,
        out_type=[
            jax.ShapeDtypeStruct((_NR, _D), jnp.float32),
            jax.ShapeDtypeStruct((_NR, 128), jnp.float32),
        ],
        mesh=mesh,
        scratch_types=[
            pltpu.VMEM((_CW, _D), jnp.float32),
            pltpu.VMEM((_CW, 128), jnp.float32),
            pltpu.VMEM((_CW, 128), jnp.float32),
            pltpu.VMEM((_CW,), jnp.int32),
            pltpu.VMEM((_CW,), jnp.int32),
            pltpu.SemaphoreType.DMA,
        ],
    )
    def dispatch(x_hbm, d_hbm, wbc_hbm, xs_hbm, rww_hbm,
                 xrows, w0_v, w1_v, d0_v, d1_v, sem):
        wid = lax.axis_index("s") * 2 + lax.axis_index("c")
        base = wid * _CW
        pltpu.sync_copy(x_hbm.at[pl.ds(base, _CW), :], xrows)
        pltpu.sync_copy(d_hbm.at[0, pl.ds(base, _CW)], d0_v)
        pltpu.sync_copy(d_hbm.at[1, pl.ds(base, _CW)], d1_v)
        pltpu.sync_copy(wbc_hbm.at[0, pl.ds(base, _CW), :], w0_v)
        pltpu.sync_copy(wbc_hbm.at[1, pl.ds(base, _CW), :], w1_v)
        pltpu.async_copy(xrows, xs_hbm.at[d0_v], sem).wait()
        pltpu.async_copy(xrows, xs_hbm.at[d1_v], sem).wait()
        pltpu.async_copy(w0_v, rww_hbm.at[d0_v], sem).wait()
        pltpu.async_copy(w1_v, rww_hbm.at[d1_v], sem).wait()

    return dispatch


def _make_combine():
    mesh = plsc.VectorSubcoreMesh(core_axis_name="c", subcore_axis_name="s")

    @functools.partial(
        pl.kernel,
        out_type=jax.ShapeDtypeStruct((_T, _D), jnp.float32),
        mesh=mesh,
        scratch_types=[
            pltpu.VMEM((_SUB, _D), jnp.float32),
            pltpu.VMEM((_SUB, _D), jnp.float32),
            pltpu.VMEM((_SUB, _D), jnp.float32),
            pltpu.VMEM((_SUB,), jnp.int32),
            pltpu.VMEM((_SUB,), jnp.int32),
            pltpu.SemaphoreType.DMA,
        ],
    )
    def combine(x_hbm, d_hbm, ysw_hbm, out_hbm, xv, g0, g1, d0_v, d1_v, sem):
        wid = lax.axis_index("s") * 2 + lax.axis_index("c")
        for s in range(_CW // _SUB):
            base = wid * _CW + s * _SUB
            pltpu.sync_copy(x_hbm.at[pl.ds(base, _SUB), :], xv)
            pltpu.sync_copy(d_hbm.at[0, pl.ds(base, _SUB)], d0_v)
            pltpu.sync_copy(d_hbm.at[1, pl.ds(base, _SUB)], d1_v)
            pltpu.async_copy(ysw_hbm.at[d0_v], g0, sem).wait()
            pltpu.async_copy(ysw_hbm.at[d1_v], g1, sem).wait()

            def body(j, carry):
                for c in range(_D // 16):
                    col = pl.ds(c * 16, 16)
                    xv[j, col] = xv[j, col] + g0[j, col] + g1[j, col]
                return carry

            lax.fori_loop(0, _SUB, body, 0)
            pltpu.sync_copy(xv, out_hbm.at[pl.ds(base, _SUB), :])

    return combine


def kernel(x, expert_indices, expert_weights, W1, b1, W2, b2):
    xf = x.reshape(_T, _D)
    idx_eo = expert_indices.reshape(_T, _K).T
    w_eo = expert_weights.reshape(_T, _K).T
    w_bc = jnp.broadcast_to(w_eo[:, :, None], (_K, _T, 128))

    d_eo, meta = pl.pallas_call(
        _route_body,
        grid=(1,),
        in_specs=[
            pl.BlockSpec((_K, _T), lambda i: (0, 0)),
        ],
        out_specs=[
            pl.BlockSpec((_K, _T), lambda i: (0, 0)),
            pl.BlockSpec((4, _G), lambda i: (0, 0)),
        ],
        out_shape=[
            jax.ShapeDtypeStruct((_K, _T), jnp.int32),
            jax.ShapeDtypeStruct((4, _G), jnp.int32),
        ],
    )(idx_eo)

    xs, roww = _make_dispatch()(xf, d_eo, w_bc)

    ysw = pl.pallas_call(
        _ffn_body,
        grid_spec=pltpu.PrefetchScalarGridSpec(
            num_scalar_prefetch=1,
            grid=(_G,),
            in_specs=[
                pl.BlockSpec((_TR, _D), lambda i, m: (i, 0)),
                pl.BlockSpec((1, 1, _H), lambda i, m: (m[0, i], 0, 0)),
                pl.BlockSpec((1, 1, _D), lambda i, m: (m[0, i], 0, 0)),
                pl.BlockSpec((_TR, 128), lambda i, m: (i, 0)),
                pl.BlockSpec(memory_space=pl.ANY),
                pl.BlockSpec(memory_space=pl.ANY),
            ],
            out_specs=pl.BlockSpec((_TR, _D), lambda i, m: (i, 0)),
            scratch_shapes=[
                pltpu.VMEM((_D, _H), jnp.float32),
                pltpu.VMEM((_H, _D), jnp.float32),
                pltpu.VMEM((_D, _H), jnp.float32),
                pltpu.VMEM((_H, _D), jnp.float32),
                pltpu.SemaphoreType.DMA((2, 2)),
            ],
        ),
        out_shape=jax.ShapeDtypeStruct((_NR, _D), jnp.float32),
    )(meta, xs, b1.reshape(_E, 1, _H), b2.reshape(_E, 1, _D), roww, W1, W2)

    out = _make_combine()(xf, d_eo, ysw)
    return out.reshape(_B, _T, _D)


# speedup vs baseline: 1.1260x; 1.1260x over previous
import functools

import jax
import jax.numpy as jnp
from jax import lax
from jax.experimental import pallas as pl
from jax.experimental.pallas import tpu as pltpu
from jax.experimental.pallas import tpu_sc as plsc

_B, _T, _D, _E, _K = 1, 2048, 768, 8, 2
_H = 4 * _D
_NP = _T * _K
_TR = 128
_NR = _NP + _E * _TR
_G = _NR // _TR
_NW = 32
_CW = _T // _NW
_SUB = 32


def _gelu(x):
    c = 0.7978845608028654
    t = jnp.tanh(c * (x + 0.044715 * x * x * x))
    return 0.5 * x * (1.0 + t)


def _cumsum_rows(a):
    n = a.shape[1]
    sh = 1
    while sh < n:
        z = jnp.zeros(a.shape[:1] + (sh,), a.dtype)
        a = a + jnp.concatenate([z, a[:, :-sh]], axis=1)
        sh *= 2
    return a


def _route_body(idx_ref, d_ref, meta_ref):
    ee = lax.broadcasted_iota(jnp.int32, (_E, 1), 0)
    m0 = (idx_ref[0:1, :] == ee).astype(jnp.float32)
    m1 = (idx_ref[1:2, :] == ee).astype(jnp.float32)
    inc0 = _cumsum_rows(m0)
    inc1 = _cumsum_rows(m1) + inc0[:, _T - 1:_T]
    counts = inc1[:, _T - 1:_T]
    padded = jnp.ceil(counts * (1.0 / _TR)) * float(_TR)
    c = padded
    sh = 1
    while sh < _E:
        z = jnp.zeros((sh, 1), jnp.float32)
        c = c + jnp.concatenate([z, c[:-sh, :]], axis=0)
        sh *= 2
    starts = c - padded
    d0 = jnp.sum(m0 * (starts + inc0), axis=0, keepdims=True) - 1.0
    d1 = jnp.sum(m1 * (starts + inc1), axis=0, keepdims=True) - 1.0
    d_ref[...] = jnp.concatenate([d0, d1], axis=0).astype(jnp.int32)

    ends = starts + padded
    tpos = (lax.broadcasted_iota(jnp.int32, (1, _G), 1)
            .astype(jnp.float32) * float(_TR))
    neid = jnp.sum((tpos >= ends).astype(jnp.float32), axis=0, keepdims=True)
    eid = jnp.clip(neid, 0, _E - 1)
    prev = jnp.concatenate([jnp.full((1, 1), -1.0, jnp.float32),
                            eid[:, :-1]], axis=1)
    first = (eid != prev).astype(jnp.float32)
    runidx = _cumsum_rows(first) - 1.0
    parity = runidx - 2.0 * jnp.floor(runidx * 0.5)
    nruns = jnp.sum(first, axis=1, keepdims=True)
    rr = lax.broadcasted_iota(jnp.int32, (_G, 1), 0).astype(jnp.float32)
    is_start = first * (runidx + 1.0)
    reid = jnp.sum(jnp.where(rr + 1.0 == is_start, eid, 0.0), axis=1,
                   keepdims=True)
    nxt = jnp.sum(jnp.where(rr == runidx + 1.0, reid, 0.0), axis=0,
                  keepdims=True)
    nxt = jnp.where(runidx + 1.0 < nruns, nxt, -1.0)
    meta = jnp.concatenate([eid, first, parity, nxt], axis=0)
    meta_ref[...] = meta.astype(jnp.int32)


def _ffn_body(meta_sref, xs_ref, b1_ref, b2_ref, rw_ref, W1_any, W2_any,
              out_ref, w1a, w2a, w1b, w2b, sems):
    i = pl.program_id(0)
    eid = meta_sref[0, i]
    first = meta_sref[1, i] == 1
    par = meta_sref[2, i]
    nei = meta_sref[3, i]

    @pl.when(i == 0)
    def _():
        pltpu.make_async_copy(W1_any.at[eid], w1a, sems.at[0, 0]).start()
        pltpu.make_async_copy(W2_any.at[eid], w2a, sems.at[0, 1]).start()

    @pl.when(first)
    def _():
        @pl.when(par == 0)
        def _():
            pltpu.make_async_copy(W1_any.at[eid], w1a, sems.at[0, 0]).wait()
            pltpu.make_async_copy(W2_any.at[eid], w2a, sems.at[0, 1]).wait()

        @pl.when(par == 1)
        def _():
            pltpu.make_async_copy(W1_any.at[eid], w1b, sems.at[1, 0]).wait()
            pltpu.make_async_copy(W2_any.at[eid], w2b, sems.at[1, 1]).wait()

        @pl.when(nei >= 0)
        def _():
            @pl.when(par == 0)
            def _():
                pltpu.make_async_copy(W1_any.at[nei], w1b,
                                      sems.at[1, 0]).start()
                pltpu.make_async_copy(W2_any.at[nei], w2b,
                                      sems.at[1, 1]).start()

            @pl.when(par == 1)
            def _():
                pltpu.make_async_copy(W1_any.at[nei], w1a,
                                      sems.at[0, 0]).start()
                pltpu.make_async_copy(W2_any.at[nei], w2a,
                                      sems.at[0, 1]).start()

    xb = xs_ref[...]
    rw = rw_ref[:, 0:1]

    @pl.when(par == 0)
    def _():
        h = jnp.dot(xb, w1a[...], preferred_element_type=jnp.float32)
        h = _gelu(h + b1_ref[0])
        y = jnp.dot(h, w2a[...], preferred_element_type=jnp.float32)
        out_ref[...] = (y + b2_ref[0]) * rw

    @pl.when(par == 1)
    def _():
        h = jnp.dot(xb, w1b[...], preferred_element_type=jnp.float32)
        h = _gelu(h + b1_ref[0])
        y = jnp.dot(h, w2b[...], preferred_element_type=jnp.float32)
        out_ref[...] = (y + b2_ref[0]) * rw


def _make_dispatch():
    mesh = plsc.VectorSubcoreMesh(core_axis_name="c", subcore_axis_name="s")

    @functools.partial(
        pl.kernel,
        out_type=[
            jax.ShapeDtypeStruct((_NR, _D), jnp.float32),
            jax.ShapeDtypeStruct((_NR, 128), jnp.float32),
        ],
        mesh=mesh,
        scratch_types=[
            pltpu.VMEM((_CW, _D), jnp.float32),
            pltpu.VMEM((_CW, 128), jnp.float32),
            pltpu.VMEM((_CW, 128), jnp.float32),
            pltpu.VMEM((_CW,), jnp.int32),
            pltpu.VMEM((_CW,), jnp.int32),
            pltpu.SemaphoreType.DMA,
        ],
    )
    def dispatch(x_hbm, d_hbm, wbc_hbm, xs_hbm, rww_hbm,
                 xrows, w0_v, w1_v, d0_v, d1_v, sem):
        wid = lax.axis_index("s") * 2 + lax.axis_index("c")
        base = wid * _CW
        pltpu.sync_copy(x_hbm.at[pl.ds(base, _CW), :], xrows)
        pltpu.sync_copy(d_hbm.at[0, pl.ds(base, _CW)], d0_v)
        pltpu.sync_copy(d_hbm.at[1, pl.ds(base, _CW)], d1_v)
        pltpu.sync_copy(wbc_hbm.at[0, pl.ds(base, _CW), :], w0_v)
        pltpu.sync_copy(wbc_hbm.at[1, pl.ds(base, _CW), :], w1_v)
        pltpu.async_copy(xrows, xs_hbm.at[d0_v], sem).wait()
        pltpu.async_copy(xrows, xs_hbm.at[d1_v], sem).wait()
        pltpu.async_copy(w0_v, rww_hbm.at[d0_v], sem).wait()
        pltpu.async_copy(w1_v, rww_hbm.at[d1_v], sem).wait()

    return dispatch


def _make_combine():
    mesh = plsc.VectorSubcoreMesh(core_axis_name="c", subcore_axis_name="s")

    @functools.partial(
        pl.kernel,
        out_type=jax.ShapeDtypeStruct((_T, _D), jnp.float32),
        mesh=mesh,
        scratch_types=[
            pltpu.VMEM((_SUB, _D), jnp.float32),
            pltpu.VMEM((_SUB, _D), jnp.float32),
            pltpu.VMEM((_SUB, _D), jnp.float32),
            pltpu.VMEM((_SUB,), jnp.int32),
            pltpu.VMEM((_SUB,), jnp.int32),
            pltpu.SemaphoreType.DMA,
        ],
    )
    def combine(x_hbm, d_hbm, ysw_hbm, out_hbm, xv, g0, g1, d0_v, d1_v, sem):
        wid = lax.axis_index("s") * 2 + lax.axis_index("c")
        for s in range(_CW // _SUB):
            base = wid * _CW + s * _SUB
            pltpu.sync_copy(x_hbm.at[pl.ds(base, _SUB), :], xv)
            pltpu.sync_copy(d_hbm.at[0, pl.ds(base, _SUB)], d0_v)
            pltpu.sync_copy(d_hbm.at[1, pl.ds(base, _SUB)], d1_v)
            pltpu.async_copy(ysw_hbm.at[d0_v], g0, sem).wait()
            pltpu.async_copy(ysw_hbm.at[d1_v], g1, sem).wait()

            def body(j, carry):
                for c in range(_D // 16):
                    col = pl.ds(c * 16, 16)
                    xv[j, col] = xv[j, col] + g0[j, col] + g1[j, col]
                return carry

            lax.fori_loop(0, _SUB, body, 0)
            pltpu.sync_copy(xv, out_hbm.at[pl.ds(base, _SUB), :])

    return combine


def kernel(x, expert_indices, expert_weights, W1, b1, W2, b2):
    xf = x.reshape(_T, _D)
    idx_eo = expert_indices.reshape(_T, _K).T
    w_eo = expert_weights.reshape(_T, _K).T
    w_bc = jnp.broadcast_to(w_eo[:, :, None], (_K, _T, 128))

    d_eo, meta = pl.pallas_call(
        _route_body,
        grid=(1,),
        in_specs=[
            pl.BlockSpec((_K, _T), lambda i: (0, 0)),
        ],
        out_specs=[
            pl.BlockSpec((_K, _T), lambda i: (0, 0)),
            pl.BlockSpec((4, _G), lambda i: (0, 0)),
        ],
        out_shape=[
            jax.ShapeDtypeStruct((_K, _T), jnp.int32),
            jax.ShapeDtypeStruct((4, _G), jnp.int32),
        ],
    )(idx_eo)

    xs, roww = _make_dispatch()(xf, d_eo, w_bc)

    ysw = pl.pallas_call(
        _ffn_body,
        grid_spec=pltpu.PrefetchScalarGridSpec(
            num_scalar_prefetch=1,
            grid=(_G,),
            in_specs=[
                pl.BlockSpec((_TR, _D), lambda i, m: (i, 0)),
                pl.BlockSpec((1, 1, _H), lambda i, m: (m[0, i], 0, 0)),
                pl.BlockSpec((1, 1, _D), lambda i, m: (m[0, i], 0, 0)),
                pl.BlockSpec((_TR, 128), lambda i, m: (i, 0)),
                pl.BlockSpec(memory_space=pl.ANY),
                pl.BlockSpec(memory_space=pl.ANY),
            ],
            out_specs=pl.BlockSpec((_TR, _D), lambda i, m: (i, 0)),
            scratch_shapes=[
                pltpu.VMEM((_D, _H), jnp.float32),
                pltpu.VMEM((_H, _D), jnp.float32),
                pltpu.VMEM((_D, _H), jnp.float32),
                pltpu.VMEM((_H, _D), jnp.float32),
                pltpu.SemaphoreType.DMA((2, 2)),
            ],
        ),
        out_shape=jax.ShapeDtypeStruct((_NR, _D), jnp.float32),
    )(meta, xs, b1.reshape(_E, 1, _H), b2.reshape(_E, 1, _D), roww, W1, W2)

    out = _make_combine()(xf, d_eo, ysw)
    return out.reshape(_B, _T, _D)


# speedup vs baseline: 1.1553x; 1.0260x over previous
import functools

import jax
import jax.numpy as jnp
from jax import lax
from jax.experimental import pallas as pl
from jax.experimental.pallas import tpu as pltpu
from jax.experimental.pallas import tpu_sc as plsc

_B, _T, _D, _E, _K = 1, 2048, 768, 8, 2
_H = 4 * _D
_NP = _T * _K
_TR = 128
_NR = _NP + _E * _TR
_G = _NR // _TR
_NW = 32
_CW = _T // _NW
_SUB = 32


def _gelu(x):
    c = 0.7978845608028654
    t = jnp.tanh(c * (x + 0.044715 * x * x * x))
    return 0.5 * x * (1.0 + t)


def _cumsum_rows(a):
    n = a.shape[1]
    sh = 1
    while sh < n:
        z = jnp.zeros(a.shape[:1] + (sh,), a.dtype)
        a = a + jnp.concatenate([z, a[:, :-sh]], axis=1)
        sh *= 2
    return a


def _route_body(idx_ref, d_ref, meta_ref):
    ee = lax.broadcasted_iota(jnp.int32, (_E, 1), 0)
    m0 = (idx_ref[0:1, :] == ee).astype(jnp.float32)
    m1 = (idx_ref[1:2, :] == ee).astype(jnp.float32)
    inc0 = _cumsum_rows(m0)
    inc1 = _cumsum_rows(m1) + inc0[:, _T - 1:_T]
    counts = inc1[:, _T - 1:_T]
    padded = jnp.ceil(counts * (1.0 / _TR)) * float(_TR)
    c = padded
    sh = 1
    while sh < _E:
        z = jnp.zeros((sh, 1), jnp.float32)
        c = c + jnp.concatenate([z, c[:-sh, :]], axis=0)
        sh *= 2
    starts = c - padded
    d0 = jnp.sum(m0 * (starts + inc0), axis=0, keepdims=True) - 1.0
    d1 = jnp.sum(m1 * (starts + inc1), axis=0, keepdims=True) - 1.0
    d_ref[...] = jnp.concatenate([d0, d1], axis=0).astype(jnp.int32)

    ends = starts + padded
    tpos = (lax.broadcasted_iota(jnp.int32, (1, _G), 1)
            .astype(jnp.float32) * float(_TR))
    neid = jnp.sum((tpos >= ends).astype(jnp.float32), axis=0, keepdims=True)
    eid = jnp.clip(neid, 0, _E - 1)
    prev = jnp.concatenate([jnp.full((1, 1), -1.0, jnp.float32),
                            eid[:, :-1]], axis=1)
    first = (eid != prev).astype(jnp.float32)
    runidx = _cumsum_rows(first) - 1.0
    parity = runidx - 2.0 * jnp.floor(runidx * 0.5)
    nruns = jnp.sum(first, axis=1, keepdims=True)
    rr = lax.broadcasted_iota(jnp.int32, (_G, 1), 0).astype(jnp.float32)
    is_start = first * (runidx + 1.0)
    reid = jnp.sum(jnp.where(rr + 1.0 == is_start, eid, 0.0), axis=1,
                   keepdims=True)
    nxt = jnp.sum(jnp.where(rr == runidx + 1.0, reid, 0.0), axis=0,
                  keepdims=True)
    nxt = jnp.where(runidx + 1.0 < nruns, nxt, -1.0)
    meta = jnp.concatenate([eid, first, parity, nxt], axis=0)
    meta_ref[...] = meta.astype(jnp.int32)


def _ffn_body(meta_sref, xs_ref, b1_ref, b2_ref, rw_ref, W1_any, W2_any,
              out_ref, w1a, w2a, w1b, w2b, sems):
    i = pl.program_id(0)
    eid = meta_sref[0, i]
    first = meta_sref[1, i] == 1
    par = meta_sref[2, i]
    nei = meta_sref[3, i]

    @pl.when(i == 0)
    def _():
        pltpu.make_async_copy(W1_any.at[eid], w1a, sems.at[0, 0]).start()
        pltpu.make_async_copy(W2_any.at[eid], w2a, sems.at[0, 1]).start()

    @pl.when(first)
    def _():
        @pl.when(par == 0)
        def _():
            pltpu.make_async_copy(W1_any.at[eid], w1a, sems.at[0, 0]).wait()
            pltpu.make_async_copy(W2_any.at[eid], w2a, sems.at[0, 1]).wait()

        @pl.when(par == 1)
        def _():
            pltpu.make_async_copy(W1_any.at[eid], w1b, sems.at[1, 0]).wait()
            pltpu.make_async_copy(W2_any.at[eid], w2b, sems.at[1, 1]).wait()

        @pl.when(nei >= 0)
        def _():
            @pl.when(par == 0)
            def _():
                pltpu.make_async_copy(W1_any.at[nei], w1b,
                                      sems.at[1, 0]).start()
                pltpu.make_async_copy(W2_any.at[nei], w2b,
                                      sems.at[1, 1]).start()

            @pl.when(par == 1)
            def _():
                pltpu.make_async_copy(W1_any.at[nei], w1a,
                                      sems.at[0, 0]).start()
                pltpu.make_async_copy(W2_any.at[nei], w2a,
                                      sems.at[0, 1]).start()

    xb = xs_ref[...]
    rw = rw_ref[:, 0:1]

    @pl.when(par == 0)
    def _():
        h = jnp.dot(xb, w1a[...], preferred_element_type=jnp.float32)
        h = _gelu(h + b1_ref[0])
        y = jnp.dot(h, w2a[...], preferred_element_type=jnp.float32)
        out_ref[...] = (y + b2_ref[0]) * rw

    @pl.when(par == 1)
    def _():
        h = jnp.dot(xb, w1b[...], preferred_element_type=jnp.float32)
        h = _gelu(h + b1_ref[0])
        y = jnp.dot(h, w2b[...], preferred_element_type=jnp.float32)
        out_ref[...] = (y + b2_ref[0]) * rw


def _make_dispatch():
    mesh = plsc.VectorSubcoreMesh(core_axis_name="c", subcore_axis_name="s")

    @functools.partial(
        pl.kernel,
        out_type=[
            jax.ShapeDtypeStruct((_NR, _D), jnp.float32),
            jax.ShapeDtypeStruct((_NR, 128), jnp.float32),
        ],
        mesh=mesh,
        scratch_types=[
            pltpu.VMEM((_CW, _D), jnp.float32),
            pltpu.VMEM((_CW, 128), jnp.float32),
            pltpu.VMEM((_CW, 128), jnp.float32),
            pltpu.VMEM((_CW,), jnp.int32),
            pltpu.VMEM((_CW,), jnp.int32),
            pltpu.SemaphoreType.DMA,
            pltpu.SemaphoreType.DMA,
            pltpu.SemaphoreType.DMA,
            pltpu.SemaphoreType.DMA,
        ],
    )
    def dispatch(x_hbm, d_hbm, wbc_hbm, xs_hbm, rww_hbm,
                 xrows, w0_v, w1_v, d0_v, d1_v, s0, s1, s2, s3):
        wid = lax.axis_index("s") * 2 + lax.axis_index("c")
        base = wid * _CW
        pltpu.sync_copy(d_hbm.at[0, pl.ds(base, _CW)], d0_v)
        pltpu.sync_copy(d_hbm.at[1, pl.ds(base, _CW)], d1_v)
        cx = pltpu.async_copy(x_hbm.at[pl.ds(base, _CW), :], xrows, s0)
        cw0 = pltpu.async_copy(wbc_hbm.at[0, pl.ds(base, _CW), :], w0_v, s1)
        cw1 = pltpu.async_copy(wbc_hbm.at[1, pl.ds(base, _CW), :], w1_v, s2)
        cw0.wait()
        cw1.wait()
        a2 = pltpu.async_copy(w0_v, rww_hbm.at[d0_v], s1)
        a3 = pltpu.async_copy(w1_v, rww_hbm.at[d1_v], s2)
        cx.wait()
        a0 = pltpu.async_copy(xrows, xs_hbm.at[d0_v], s0)
        a1 = pltpu.async_copy(xrows, xs_hbm.at[d1_v], s3)
        a2.wait()
        a3.wait()
        a0.wait()
        a1.wait()

    return dispatch


def _make_combine():
    mesh = plsc.VectorSubcoreMesh(core_axis_name="c", subcore_axis_name="s")

    @functools.partial(
        pl.kernel,
        out_type=jax.ShapeDtypeStruct((_T, _D), jnp.float32),
        mesh=mesh,
        scratch_types=[
            pltpu.VMEM((_SUB, _D), jnp.float32),
            pltpu.VMEM((_SUB, _D), jnp.float32),
            pltpu.VMEM((_SUB, _D), jnp.float32),
            pltpu.VMEM((_SUB,), jnp.int32),
            pltpu.VMEM((_SUB,), jnp.int32),
            pltpu.SemaphoreType.DMA,
            pltpu.SemaphoreType.DMA,
            pltpu.SemaphoreType.DMA,
        ],
    )
    def combine(x_hbm, d_hbm, ysw_hbm, out_hbm, xv, g0, g1, d0_v, d1_v,
                sx, s0, s1):
        wid = lax.axis_index("s") * 2 + lax.axis_index("c")
        for sc in range(_CW // _SUB):
            base = wid * _CW + sc * _SUB
            pltpu.sync_copy(d_hbm.at[0, pl.ds(base, _SUB)], d0_v)
            pltpu.sync_copy(d_hbm.at[1, pl.ds(base, _SUB)], d1_v)
            cx = pltpu.async_copy(x_hbm.at[pl.ds(base, _SUB), :], xv, sx)
            c0 = pltpu.async_copy(ysw_hbm.at[d0_v], g0, s0)
            c1 = pltpu.async_copy(ysw_hbm.at[d1_v], g1, s1)
            c0.wait()
            c1.wait()
            cx.wait()

            def body(j, carry):
                for c in range(_D // 16):
                    col = pl.ds(c * 16, 16)
                    xv[j, col] = xv[j, col] + g0[j, col] + g1[j, col]
                return carry

            lax.fori_loop(0, _SUB, body, 0)
            pltpu.sync_copy(xv, out_hbm.at[pl.ds(base, _SUB), :])

    return combine


def kernel(x, expert_indices, expert_weights, W1, b1, W2, b2):
    xf = x.reshape(_T, _D)
    idx_eo = expert_indices.reshape(_T, _K).T
    w_eo = expert_weights.reshape(_T, _K).T
    w_bc = jnp.broadcast_to(w_eo[:, :, None], (_K, _T, 128))

    d_eo, meta = pl.pallas_call(
        _route_body,
        grid=(1,),
        in_specs=[
            pl.BlockSpec((_K, _T), lambda i: (0, 0)),
        ],
        out_specs=[
            pl.BlockSpec((_K, _T), lambda i: (0, 0)),
            pl.BlockSpec((4, _G), lambda i: (0, 0)),
        ],
        out_shape=[
            jax.ShapeDtypeStruct((_K, _T), jnp.int32),
            jax.ShapeDtypeStruct((4, _G), jnp.int32),
        ],
    )(idx_eo)

    xs, roww = _make_dispatch()(xf, d_eo, w_bc)

    ysw = pl.pallas_call(
        _ffn_body,
        grid_spec=pltpu.PrefetchScalarGridSpec(
            num_scalar_prefetch=1,
            grid=(_G,),
            in_specs=[
                pl.BlockSpec((_TR, _D), lambda i, m: (i, 0)),
                pl.BlockSpec((1, 1, _H), lambda i, m: (m[0, i], 0, 0)),
                pl.BlockSpec((1, 1, _D), lambda i, m: (m[0, i], 0, 0)),
                pl.BlockSpec((_TR, 128), lambda i, m: (i, 0)),
                pl.BlockSpec(memory_space=pl.ANY),
                pl.BlockSpec(memory_space=pl.ANY),
            ],
            out_specs=pl.BlockSpec((_TR, _D), lambda i, m: (i, 0)),
            scratch_shapes=[
                pltpu.VMEM((_D, _H), jnp.float32),
                pltpu.VMEM((_H, _D), jnp.float32),
                pltpu.VMEM((_D, _H), jnp.float32),
                pltpu.VMEM((_H, _D), jnp.float32),
                pltpu.SemaphoreType.DMA((2, 2)),
            ],
        ),
        out_shape=jax.ShapeDtypeStruct((_NR, _D), jnp.float32),
    )(meta, xs, b1.reshape(_E, 1, _H), b2.reshape(_E, 1, _D), roww, W1, W2)

    out = _make_combine()(xf, d_eo, ysw)
    return out.reshape(_B, _T, _D)
